# baseline (speedup 1.0000x reference)
import functools

import jax
import jax.numpy as jnp
from jax import lax
from jax.experimental import pallas as pl
from jax.experimental.pallas import tpu as pltpu
from jax.experimental.pallas import tpu_sc as plsc

_N_GRAPH, _N_NODE, _D = 256, 128, 512
_NS = 16
_ROWS = _N_GRAPH // _NS
_H = _ROWS // 2
_L = 16


def _gather_body(xf, root, out, root_v, idx_v, rows_a, rows_b, s1, s2, s3, s4):
    s = lax.axis_index("s")
    base = s * _ROWS
    pltpu.sync_copy(root.at[pl.ds(base, _ROWS)], root_v)
    idx_v[...] = root_v[...] + (lax.iota(jnp.int32, _L) + base) * _N_NODE
    cp_a = pltpu.async_copy(xf.at[idx_v.at[pl.ds(0, _H)]], rows_a, s1)
    cp_b = pltpu.async_copy(xf.at[idx_v.at[pl.ds(_H, _H)]], rows_b, s2)
    cp_a.wait()
    out_a = pltpu.async_copy(rows_a, out.at[pl.ds(base, _H)], s3)
    cp_b.wait()
    out_b = pltpu.async_copy(rows_b, out.at[pl.ds(base + _H, _H)], s4)
    out_a.wait()
    out_b.wait()


def kernel(x, x_mask, root_n_id, attn):
    del x_mask, attn
    xf = x.reshape(-1, _D)
    root = root_n_id.astype(jnp.int32)
    mesh = plsc.VectorSubcoreMesh(
        core_axis_name="c", subcore_axis_name="s", num_cores=1
    )
    f = functools.partial(
        pl.kernel,
        mesh=mesh,
        out_type=jax.ShapeDtypeStruct((_N_GRAPH, _D), jnp.float32),
        scratch_types=[
            pltpu.VMEM((_L,), jnp.int32),
            pltpu.VMEM((_L,), jnp.int32),
            pltpu.VMEM((_H, _D), jnp.float32),
            pltpu.VMEM((_H, _D), jnp.float32),
            pltpu.SemaphoreType.DMA,
            pltpu.SemaphoreType.DMA,
            pltpu.SemaphoreType.DMA,
            pltpu.SemaphoreType.DMA,
        ],
    )(_gather_body)
    return f(xf, root)
